# Initial kernel scaffold; baseline (speedup 1.0000x reference)
import functools

import jax
import jax.numpy as jnp
from jax import lax
from jax.experimental import pallas as pl
from jax.experimental.pallas import tpu as pltpu
from jax.experimental.pallas import tpu_sc as plsc

_N = 10000
_E = 320000
_FIN = 128
_HID = 256
_NCLS = 64

_NC = 2
_NS = 16
_NW = _NC * _NS
_EPW = _E // _NW
_BLK = 80
_NBLK = _EPW // _BLK
_RPW = _N // _NS
_ZR = 125
_DEGW = 16

_mesh = plsc.VectorSubcoreMesh(
    core_axis_name="c", subcore_axis_name="s", num_cores=_NC, num_subcores=_NS
)


def _deg_body(dst_hbm, degp_hbm, dstv, onesv, zv, acc_sh):
  c = lax.axis_index("c")
  s = lax.axis_index("s")
  w = c * _NS + s

  def _init(i, carry):
    onesv[i, :] = jnp.ones((_DEGW,), jnp.float32)
    return carry

  lax.fori_loop(0, _BLK, _init, 0)

  def _zinit(i, carry):
    zv[i, :] = jnp.zeros((_DEGW,), jnp.float32)
    return carry

  lax.fori_loop(0, _RPW, _zinit, 0)
  pltpu.sync_copy(zv, acc_sh.at[pl.ds(s * _RPW, _RPW)])
  pltpu.sync_copy(dst_hbm.at[w], dstv)
  plsc.subcore_barrier()

  def _step(j, carry):
    pltpu.sync_copy(onesv, acc_sh.at[dstv.at[j]], add=True)
    return carry

  lax.fori_loop(0, _NBLK, _step, 0)
  plsc.subcore_barrier()
  pltpu.sync_copy(acc_sh.at[pl.ds(s * _RPW, _RPW)],
                  degp_hbm.at[c, pl.ds(s * _RPW, _RPW)])


_deg_kernel = functools.partial(
    pl.kernel,
    out_type=jax.ShapeDtypeStruct((_NC, _N, _DEGW), jnp.float32),
    mesh=_mesh,
    scratch_types=[
        pltpu.VMEM((_NBLK, _BLK), jnp.int32),
        pltpu.VMEM((_BLK, _DEGW), jnp.float32),
        pltpu.VMEM((_RPW, _DEGW), jnp.float32),
        pltpu.VMEM_SHARED((_N, _DEGW), jnp.float32),
    ],
)(_deg_body)


def _make_agg(feat):
  def _agg_body(xs_hbm, src_hbm, dst_hbm, aggp_hbm, srcv, dstv, rows, zv,
                acc_sh):
    c = lax.axis_index("c")
    s = lax.axis_index("s")
    w = c * _NS + s

    def _zinit(i, carry):
      for f in range(feat // 16):
        zv[i, pl.ds(f * 16, 16)] = jnp.zeros((16,), jnp.float32)
      return carry

    lax.fori_loop(0, _ZR, _zinit, 0)
    for r in range(_RPW // _ZR):
      pltpu.sync_copy(zv, acc_sh.at[pl.ds(s * _RPW + r * _ZR, _ZR)])
    pltpu.sync_copy(src_hbm.at[w], srcv)
    pltpu.sync_copy(dst_hbm.at[w], dstv)
    plsc.subcore_barrier()

    def _step(j, carry):
      pltpu.sync_copy(xs_hbm.at[srcv.at[j]], rows)
      pltpu.sync_copy(rows, acc_sh.at[dstv.at[j]], add=True)
      return carry

    lax.fori_loop(0, _NBLK, _step, 0)
    plsc.subcore_barrier()
    pltpu.sync_copy(acc_sh.at[pl.ds(s * _RPW, _RPW)],
                    aggp_hbm.at[c, pl.ds(s * _RPW, _RPW)])

  return functools.partial(
      pl.kernel,
      out_type=jax.ShapeDtypeStruct((_NC, _N, feat), jnp.float32),
      mesh=_mesh,
      scratch_types=[
          pltpu.VMEM((_NBLK, _BLK), jnp.int32),
          pltpu.VMEM((_NBLK, _BLK), jnp.int32),
          pltpu.VMEM((_BLK, feat), jnp.float32),
          pltpu.VMEM((_ZR, feat), jnp.float32),
          pltpu.VMEM_SHARED((_N, feat), jnp.float32),
      ],
  )(_agg_body)


_agg128 = _make_agg(_FIN)
_agg64 = _make_agg(_NCLS)


_BN = 1000
_GN = _N // _BN


def _dinv_from(degp_ref):
  deg = (jnp.sum(degp_ref[0], axis=1) + jnp.sum(degp_ref[1], axis=1) + 1.0)
  return lax.rsqrt(deg)[:, None]


def _prep_body(degp_ref, x_ref, xs_ref):
  xs_ref[...] = x_ref[...] * _dinv_from(degp_ref)


_prep = pl.pallas_call(
    _prep_body,
    grid=(_GN,),
    in_specs=[
        pl.BlockSpec((_NC, _BN, _DEGW), lambda i: (0, i, 0)),
        pl.BlockSpec((_BN, _FIN), lambda i: (i, 0)),
    ],
    out_specs=pl.BlockSpec((_BN, _FIN), lambda i: (i, 0)),
    out_shape=jax.ShapeDtypeStruct((_N, _FIN), jnp.float32),
)


def _mid_body(degp_ref, aggp_ref, x_ref, w1_ref, b1_ref, w2_ref, z_ref,
              zs_ref):
  dinv = _dinv_from(degp_ref)
  y1 = dinv * (aggp_ref[0] + aggp_ref[1]) + (dinv * dinv) * x_ref[...]
  h = jnp.dot(y1, w1_ref[...], preferred_element_type=jnp.float32)
  h = jnp.maximum(h + b1_ref[...], 0.0)
  z = jnp.dot(h, w2_ref[...], preferred_element_type=jnp.float32)
  z_ref[...] = z
  zs_ref[...] = dinv * z


_mid = pl.pallas_call(
    _mid_body,
    grid=(_GN,),
    in_specs=[
        pl.BlockSpec((_NC, _BN, _DEGW), lambda i: (0, i, 0)),
        pl.BlockSpec((_NC, _BN, _FIN), lambda i: (0, i, 0)),
        pl.BlockSpec((_BN, _FIN), lambda i: (i, 0)),
        pl.BlockSpec((_FIN, _HID), lambda i: (0, 0)),
        pl.BlockSpec((1, _HID), lambda i: (0, 0)),
        pl.BlockSpec((_HID, _NCLS), lambda i: (0, 0)),
    ],
    out_specs=[
        pl.BlockSpec((_BN, _NCLS), lambda i: (i, 0)),
        pl.BlockSpec((_BN, _NCLS), lambda i: (i, 0)),
    ],
    out_shape=[
        jax.ShapeDtypeStruct((_N, _NCLS), jnp.float32),
        jax.ShapeDtypeStruct((_N, _NCLS), jnp.float32),
    ],
)


def _final_body(degp_ref, aggp_ref, z_ref, b2_ref, o_ref):
  dinv = _dinv_from(degp_ref)
  y2 = (dinv * (aggp_ref[0] + aggp_ref[1]) + (dinv * dinv) * z_ref[...]
        + b2_ref[...])
  m = jnp.max(y2, axis=1, keepdims=True)
  lse = jnp.log(jnp.sum(jnp.exp(y2 - m), axis=1, keepdims=True)) + m
  o_ref[...] = y2 - lse


_final = pl.pallas_call(
    _final_body,
    grid=(_GN,),
    in_specs=[
        pl.BlockSpec((_NC, _BN, _DEGW), lambda i: (0, i, 0)),
        pl.BlockSpec((_NC, _BN, _NCLS), lambda i: (0, i, 0)),
        pl.BlockSpec((_BN, _NCLS), lambda i: (i, 0)),
        pl.BlockSpec((1, _NCLS), lambda i: (0, 0)),
    ],
    out_specs=pl.BlockSpec((_BN, _NCLS), lambda i: (i, 0)),
    out_shape=jax.ShapeDtypeStruct((_N, _NCLS), jnp.float32),
)


def kernel(x, edge_index, W1, b1, W2, b2):
  ei = edge_index.astype(jnp.int32)
  src = ei[0].reshape(_NW, _NBLK, _BLK)
  dst = ei[1].reshape(_NW, _NBLK, _BLK)

  degp = _deg_kernel(dst)
  xs = _prep(degp, x)
  aggp1 = _agg128(xs, src, dst)
  z, zs = _mid(degp, aggp1, x, W1, b1.reshape(1, _HID), W2)
  aggp2 = _agg64(zs, src, dst)
  return _final(degp, aggp2, z, b2.reshape(1, _NCLS))


# speedup vs baseline: 16.7159x; 16.7159x over previous
import functools

import jax
import jax.numpy as jnp
from jax import lax
from jax.experimental import pallas as pl
from jax.experimental.pallas import tpu as pltpu
from jax.experimental.pallas import tpu_sc as plsc

_N = 10000
_E = 320000
_FIN = 128
_HID = 256
_NCLS = 64

_NC = 2
_NS = 16
_NW = _NC * _NS
_BLK = 80
_EPT = _E // _NS
_NBLK = _EPT // _BLK
_EPW = _E // _NW
_DBLK = _EPW // _BLK
_RPT = _N // _NS
_DEGW = 16

_mesh = plsc.VectorSubcoreMesh(
    core_axis_name="c", subcore_axis_name="s", num_cores=_NC, num_subcores=_NS
)
_mesh1 = plsc.VectorSubcoreMesh(
    core_axis_name="c", subcore_axis_name="s", num_cores=1, num_subcores=_NS
)
_sc_params = pltpu.CompilerParams(use_tc_tiling_on_sc=False)


def _deg_body(dst_hbm, degp_hbm, dstv, onesv, zv, acc_sh):
  s = lax.axis_index("s")

  def _init(i, carry):
    onesv[i, :] = jnp.ones((_DEGW,), jnp.float32)
    return carry

  lax.fori_loop(0, _BLK, _init, 0)

  def _zinit(i, carry):
    zv[i, :] = jnp.zeros((_DEGW,), jnp.float32)
    return carry

  lax.fori_loop(0, _RPT, _zinit, 0)
  pltpu.sync_copy(zv, acc_sh.at[pl.ds(s * _RPT, _RPT)])
  pltpu.sync_copy(dst_hbm.at[s], dstv)
  plsc.subcore_barrier()

  def _step(j, carry):
    pltpu.sync_copy(onesv, acc_sh.at[dstv.at[j]], add=True)
    return carry

  lax.fori_loop(0, _NBLK, _step, 0)
  plsc.subcore_barrier()
  pltpu.sync_copy(acc_sh.at[pl.ds(s * _RPT, _RPT)],
                  degp_hbm.at[pl.ds(s * _RPT, _RPT)])


_deg_kernel = functools.partial(
    pl.kernel,
    out_type=jax.ShapeDtypeStruct((_N, _DEGW), jnp.float32),
    mesh=_mesh1,
    compiler_params=_sc_params,
    scratch_types=[
        pltpu.VMEM((_NBLK, _BLK), jnp.int32),
        pltpu.VMEM((_BLK, _DEGW), jnp.float32),
        pltpu.VMEM((_RPT, _DEGW), jnp.float32),
        pltpu.VMEM_SHARED((_N, _DEGW), jnp.float32),
    ],
)(_deg_body)


def _make_agg(fh):
  def _agg_body(tab_hbm, src_hbm, dst_hbm, aggp_hbm, srcv, dstv, rows, zv,
                acc_sh):
    c = lax.axis_index("c")
    s = lax.axis_index("s")

    def _zinit(i, carry):
      for f in range(fh // 16):
        zv[i, pl.ds(f * 16, 16)] = jnp.zeros((16,), jnp.float32)
      return carry

    lax.fori_loop(0, _RPT, _zinit, 0)
    pltpu.sync_copy(zv, acc_sh.at[pl.ds(s * _RPT, _RPT)])
    pltpu.sync_copy(src_hbm.at[s], srcv)
    pltpu.sync_copy(dst_hbm.at[s], dstv)
    plsc.subcore_barrier()

    def _step(j, carry):
      pltpu.sync_copy(tab_hbm.at[c].at[srcv.at[j]], rows)
      pltpu.sync_copy(rows, acc_sh.at[dstv.at[j]], add=True)
      return carry

    lax.fori_loop(0, _NBLK, _step, 0)
    plsc.subcore_barrier()
    pltpu.sync_copy(acc_sh.at[pl.ds(s * _RPT, _RPT)],
                    aggp_hbm.at[c, pl.ds(s * _RPT, _RPT)])

  return functools.partial(
      pl.kernel,
      out_type=jax.ShapeDtypeStruct((_NC, _N, fh), jnp.float32),
      mesh=_mesh,
      compiler_params=_sc_params,
      scratch_types=[
          pltpu.VMEM((_NBLK, _BLK), jnp.int32),
          pltpu.VMEM((_NBLK, _BLK), jnp.int32),
          pltpu.VMEM((_BLK, fh), jnp.float32),
          pltpu.VMEM((_RPT, fh), jnp.float32),
          pltpu.VMEM_SHARED((_N, fh), jnp.float32),
      ],
  )(_agg_body)


_agg1 = _make_agg(_FIN // 2)
_agg2 = _make_agg(_NCLS // 2)


_BN = 1000
_GN = _N // _BN
_FH1 = _FIN // 2
_FH2 = _NCLS // 2


def _dinv_from(degp_ref):
  deg = jnp.sum(degp_ref[...], axis=1) * (1.0 / _DEGW) + 1.0
  return lax.rsqrt(deg)[:, None]


def _prep_body(degp_ref, x_ref, xs2_ref):
  xs = x_ref[...] * _dinv_from(degp_ref)
  xs2_ref[0] = xs[:, :_FH1]
  xs2_ref[1] = xs[:, _FH1:]


_prep = pl.pallas_call(
    _prep_body,
    grid=(_GN,),
    in_specs=[
        pl.BlockSpec((_BN, _DEGW), lambda i: (i, 0)),
        pl.BlockSpec((_BN, _FIN), lambda i: (i, 0)),
    ],
    out_specs=pl.BlockSpec((_NC, _BN, _FH1), lambda i: (0, i, 0)),
    out_shape=jax.ShapeDtypeStruct((_NC, _N, _FH1), jnp.float32),
)


def _mid_body(degp_ref, aggp_ref, x_ref, w1_ref, b1_ref, w2_ref, z_ref,
              zs2_ref):
  dinv = _dinv_from(degp_ref)
  d2 = dinv * dinv
  y1a = dinv * aggp_ref[0] + d2 * x_ref[:, :_FH1]
  y1b = dinv * aggp_ref[1] + d2 * x_ref[:, _FH1:]
  h = (jnp.dot(y1a, w1_ref[:_FH1, :], preferred_element_type=jnp.float32)
       + jnp.dot(y1b, w1_ref[_FH1:, :], preferred_element_type=jnp.float32))
  h = jnp.maximum(h + b1_ref[...], 0.0)
  z = jnp.dot(h, w2_ref[...], preferred_element_type=jnp.float32)
  z_ref[...] = z
  zs = dinv * z
  zs2_ref[0] = zs[:, :_FH2]
  zs2_ref[1] = zs[:, _FH2:]


_mid = pl.pallas_call(
    _mid_body,
    grid=(_GN,),
    in_specs=[
        pl.BlockSpec((_BN, _DEGW), lambda i: (i, 0)),
        pl.BlockSpec((_NC, _BN, _FH1), lambda i: (0, i, 0)),
        pl.BlockSpec((_BN, _FIN), lambda i: (i, 0)),
        pl.BlockSpec((_FIN, _HID), lambda i: (0, 0)),
        pl.BlockSpec((1, _HID), lambda i: (0, 0)),
        pl.BlockSpec((_HID, _NCLS), lambda i: (0, 0)),
    ],
    out_specs=[
        pl.BlockSpec((_BN, _NCLS), lambda i: (i, 0)),
        pl.BlockSpec((_NC, _BN, _FH2), lambda i: (0, i, 0)),
    ],
    out_shape=[
        jax.ShapeDtypeStruct((_N, _NCLS), jnp.float32),
        jax.ShapeDtypeStruct((_NC, _N, _FH2), jnp.float32),
    ],
)


def _final_body(degp_ref, aggp_ref, z_ref, b2_ref, o_ref):
  dinv = _dinv_from(degp_ref)
  agg = jnp.concatenate([aggp_ref[0], aggp_ref[1]], axis=1)
  y2 = dinv * agg + (dinv * dinv) * z_ref[...] + b2_ref[...]
  m = jnp.max(y2, axis=1, keepdims=True)
  lse = jnp.log(jnp.sum(jnp.exp(y2 - m), axis=1, keepdims=True)) + m
  o_ref[...] = y2 - lse


_final = pl.pallas_call(
    _final_body,
    grid=(_GN,),
    in_specs=[
        pl.BlockSpec((_BN, _DEGW), lambda i: (i, 0)),
        pl.BlockSpec((_NC, _BN, _FH2), lambda i: (0, i, 0)),
        pl.BlockSpec((_BN, _NCLS), lambda i: (i, 0)),
        pl.BlockSpec((1, _NCLS), lambda i: (0, 0)),
    ],
    out_specs=pl.BlockSpec((_BN, _NCLS), lambda i: (i, 0)),
    out_shape=jax.ShapeDtypeStruct((_N, _NCLS), jnp.float32),
)


def kernel(x, edge_index, W1, b1, W2, b2):
  ei = edge_index.astype(jnp.int32)
  src_a = ei[0].reshape(_NS, _NBLK, _BLK)
  dst_a = ei[1].reshape(_NS, _NBLK, _BLK)

  degp = _deg_kernel(dst_a)
  xs2 = _prep(degp, x)
  aggp1 = _agg1(xs2, src_a, dst_a)
  z, zs2 = _mid(degp, aggp1, x, W1, b1.reshape(1, _HID), W2)
  aggp2 = _agg2(zs2, src_a, dst_a)
  return _final(degp, aggp2, z, b2.reshape(1, _NCLS))


# speedup vs baseline: 32.9361x; 1.9703x over previous
import functools

import jax
import jax.numpy as jnp
from jax import lax
from jax.experimental import pallas as pl
from jax.experimental.pallas import tpu as pltpu
from jax.experimental.pallas import tpu_sc as plsc

_N = 10000
_E = 320000
_FIN = 128
_HID = 256
_NCLS = 64

_NC = 2
_NS = 16
_BLK = 125
_EPT = _E // _NS
_NBLK = _EPT // _BLK
_RPT = _N // _NS
_ZR = 125
_DEGW = 16
_NRING = 4
_DWIN = 16

_mesh = plsc.VectorSubcoreMesh(
    core_axis_name="c", subcore_axis_name="s", num_cores=_NC, num_subcores=_NS
)
_mesh1 = plsc.VectorSubcoreMesh(
    core_axis_name="c", subcore_axis_name="s", num_cores=1, num_subcores=_NS
)
_sc_params = pltpu.CompilerParams(use_tc_tiling_on_sc=False)


def _deg_body(dst_hbm, degp_hbm, dstv, onesv, zv, ssem, acc_sh):
  s = lax.axis_index("s")

  def _init(i, carry):
    onesv[i, :] = jnp.ones((_DEGW,), jnp.float32)
    zv[i, :] = jnp.zeros((_DEGW,), jnp.float32)
    return carry

  lax.fori_loop(0, _ZR, _init, 0)
  for r in range(_RPT // _ZR):
    pltpu.sync_copy(zv, acc_sh.at[pl.ds(s * _RPT + r * _ZR, _ZR)])
  pltpu.sync_copy(dst_hbm.at[s], dstv)
  plsc.subcore_barrier()

  def _step(j, carry):
    @pl.when(j >= _DWIN)
    def _():
      pltpu.make_async_copy(onesv, acc_sh.at[dstv.at[j]], ssem).wait()

    pltpu.async_copy(onesv, acc_sh.at[dstv.at[j]], ssem, add=True)
    return carry

  lax.fori_loop(0, _NBLK, _step, 0)

  def _drain(j, carry):
    pltpu.make_async_copy(onesv, acc_sh.at[dstv.at[j]], ssem).wait()
    return carry

  lax.fori_loop(0, _DWIN, _drain, 0)
  plsc.subcore_barrier()
  pltpu.sync_copy(acc_sh.at[pl.ds(s * _RPT, _RPT)],
                  degp_hbm.at[pl.ds(s * _RPT, _RPT)])


_deg_kernel = functools.partial(
    pl.kernel,
    out_type=jax.ShapeDtypeStruct((_N, _DEGW), jnp.float32),
    mesh=_mesh1,
    compiler_params=_sc_params,
    scratch_types=[
        pltpu.VMEM((_NBLK, _BLK), jnp.int32),
        pltpu.VMEM((_BLK, _DEGW), jnp.float32),
        pltpu.VMEM((_ZR, _DEGW), jnp.float32),
        pltpu.SemaphoreType.DMA,
        pltpu.VMEM_SHARED((_N, _DEGW), jnp.float32),
    ],
)(_deg_body)


def _make_agg(fh):
  def _agg_body(tab_hbm, src_hbm, dst_hbm, aggp_hbm, srcv, dstv, rows, zv,
                g0, g1, g2, g3, s0, s1, s2, s3, acc_sh):
    gsems = (g0, g1, g2, g3)
    ssems = (s0, s1, s2, s3)
    c = lax.axis_index("c")
    s = lax.axis_index("s")

    def _zinit(i, carry):
      for f in range(fh // 16):
        zv[i, pl.ds(f * 16, 16)] = jnp.zeros((16,), jnp.float32)
      return carry

    lax.fori_loop(0, _ZR, _zinit, 0)
    for r in range(_RPT // _ZR):
      pltpu.sync_copy(zv, acc_sh.at[pl.ds(s * _RPT + r * _ZR, _ZR)])
    pltpu.sync_copy(src_hbm.at[s], srcv)
    pltpu.sync_copy(dst_hbm.at[s], dstv)
    plsc.subcore_barrier()

    for b in range(2):
      pltpu.async_copy(tab_hbm.at[c].at[srcv.at[b]], rows.at[b], gsems[b])

    def _outer(j4, carry):
      for b in range(_NRING):
        j = j4 * _NRING + b
        pltpu.make_async_copy(tab_hbm.at[c].at[srcv.at[j]], rows.at[b],
                              gsems[b]).wait()
        pltpu.async_copy(rows.at[b], acc_sh.at[dstv.at[j]], ssems[b],
                         add=True)
        jn = j + 2
        bn = (b + 2) % _NRING

        @pl.when(jn < _NBLK)
        def _():
          @pl.when(jn >= _NRING)
          def _():
            pltpu.make_async_copy(rows.at[bn], acc_sh.at[dstv.at[jn]],
                                  ssems[bn]).wait()

          pltpu.async_copy(tab_hbm.at[c].at[srcv.at[jn]], rows.at[bn],
                           gsems[bn])
      return carry

    lax.fori_loop(0, _NBLK // _NRING, _outer, 0)
    for b in range(_NRING):
      pltpu.make_async_copy(rows.at[b], acc_sh.at[dstv.at[0]],
                            ssems[b]).wait()
    plsc.subcore_barrier()
    pltpu.sync_copy(acc_sh.at[pl.ds(s * _RPT, _RPT)],
                    aggp_hbm.at[c, pl.ds(s * _RPT, _RPT)])

  return functools.partial(
      pl.kernel,
      out_type=jax.ShapeDtypeStruct((_NC, _N, fh), jnp.float32),
      mesh=_mesh,
      compiler_params=_sc_params,
      scratch_types=[
          pltpu.VMEM((_NBLK, _BLK), jnp.int32),
          pltpu.VMEM((_NBLK, _BLK), jnp.int32),
          pltpu.VMEM((_NRING, _BLK, fh), jnp.float32),
          pltpu.VMEM((_ZR, fh), jnp.float32),
          pltpu.SemaphoreType.DMA,
          pltpu.SemaphoreType.DMA,
          pltpu.SemaphoreType.DMA,
          pltpu.SemaphoreType.DMA,
          pltpu.SemaphoreType.DMA,
          pltpu.SemaphoreType.DMA,
          pltpu.SemaphoreType.DMA,
          pltpu.SemaphoreType.DMA,
          pltpu.VMEM_SHARED((_N, fh), jnp.float32),
      ],
  )(_agg_body)


_agg1 = _make_agg(_FIN // 2)
_agg2 = _make_agg(_NCLS // 2)


_BN = 1000
_GN = _N // _BN
_FH1 = _FIN // 2
_FH2 = _NCLS // 2


def _dinv_from(degp_ref):
  deg = jnp.sum(degp_ref[...], axis=1) * (1.0 / _DEGW) + 1.0
  return lax.rsqrt(deg)[:, None]


def _prep_body(degp_ref, x_ref, xs2_ref):
  xs = x_ref[...] * _dinv_from(degp_ref)
  xs2_ref[0] = xs[:, :_FH1]
  xs2_ref[1] = xs[:, _FH1:]


_prep = pl.pallas_call(
    _prep_body,
    grid=(_GN,),
    in_specs=[
        pl.BlockSpec((_BN, _DEGW), lambda i: (i, 0)),
        pl.BlockSpec((_BN, _FIN), lambda i: (i, 0)),
    ],
    out_specs=pl.BlockSpec((_NC, _BN, _FH1), lambda i: (0, i, 0)),
    out_shape=jax.ShapeDtypeStruct((_NC, _N, _FH1), jnp.float32),
)


def _mid_body(degp_ref, aggp_ref, x_ref, w1_ref, b1_ref, w2_ref, z_ref,
              zs2_ref):
  dinv = _dinv_from(degp_ref)
  d2 = dinv * dinv
  y1a = dinv * aggp_ref[0] + d2 * x_ref[:, :_FH1]
  y1b = dinv * aggp_ref[1] + d2 * x_ref[:, _FH1:]
  h = (jnp.dot(y1a, w1_ref[:_FH1, :], preferred_element_type=jnp.float32)
       + jnp.dot(y1b, w1_ref[_FH1:, :], preferred_element_type=jnp.float32))
  h = jnp.maximum(h + b1_ref[...], 0.0)
  z = jnp.dot(h, w2_ref[...], preferred_element_type=jnp.float32)
  z_ref[...] = z
  zs = dinv * z
  zs2_ref[0] = zs[:, :_FH2]
  zs2_ref[1] = zs[:, _FH2:]


_mid = pl.pallas_call(
    _mid_body,
    grid=(_GN,),
    in_specs=[
        pl.BlockSpec((_BN, _DEGW), lambda i: (i, 0)),
        pl.BlockSpec((_NC, _BN, _FH1), lambda i: (0, i, 0)),
        pl.BlockSpec((_BN, _FIN), lambda i: (i, 0)),
        pl.BlockSpec((_FIN, _HID), lambda i: (0, 0)),
        pl.BlockSpec((1, _HID), lambda i: (0, 0)),
        pl.BlockSpec((_HID, _NCLS), lambda i: (0, 0)),
    ],
    out_specs=[
        pl.BlockSpec((_BN, _NCLS), lambda i: (i, 0)),
        pl.BlockSpec((_NC, _BN, _FH2), lambda i: (0, i, 0)),
    ],
    out_shape=[
        jax.ShapeDtypeStruct((_N, _NCLS), jnp.float32),
        jax.ShapeDtypeStruct((_NC, _N, _FH2), jnp.float32),
    ],
)


def _final_body(degp_ref, aggp_ref, z_ref, b2_ref, o_ref):
  dinv = _dinv_from(degp_ref)
  agg = jnp.concatenate([aggp_ref[0], aggp_ref[1]], axis=1)
  y2 = dinv * agg + (dinv * dinv) * z_ref[...] + b2_ref[...]
  m = jnp.max(y2, axis=1, keepdims=True)
  lse = jnp.log(jnp.sum(jnp.exp(y2 - m), axis=1, keepdims=True)) + m
  o_ref[...] = y2 - lse


_final = pl.pallas_call(
    _final_body,
    grid=(_GN,),
    in_specs=[
        pl.BlockSpec((_BN, _DEGW), lambda i: (i, 0)),
        pl.BlockSpec((_NC, _BN, _FH2), lambda i: (0, i, 0)),
        pl.BlockSpec((_BN, _NCLS), lambda i: (i, 0)),
        pl.BlockSpec((1, _NCLS), lambda i: (0, 0)),
    ],
    out_specs=pl.BlockSpec((_BN, _NCLS), lambda i: (i, 0)),
    out_shape=jax.ShapeDtypeStruct((_N, _NCLS), jnp.float32),
)


def kernel(x, edge_index, W1, b1, W2, b2):
  ei = edge_index.astype(jnp.int32)
  src_a = ei[0].reshape(_NS, _NBLK, _BLK)
  dst_a = ei[1].reshape(_NS, _NBLK, _BLK)

  degp = _deg_kernel(dst_a)
  xs2 = _prep(degp, x)
  aggp1 = _agg1(xs2, src_a, dst_a)
  z, zs2 = _mid(degp, aggp1, x, W1, b1.reshape(1, _HID), W2)
  aggp2 = _agg2(zs2, src_a, dst_a)
  return _final(degp, aggp2, z, b2.reshape(1, _NCLS))


# speedup vs baseline: 39.9486x; 1.2129x over previous
import functools

import jax
import jax.numpy as jnp
from jax import lax
from jax.experimental import pallas as pl
from jax.experimental.pallas import tpu as pltpu
from jax.experimental.pallas import tpu_sc as plsc

_N = 10000
_E = 320000
_FIN = 128
_HID = 256
_NCLS = 64

_NC = 2
_NS = 16
_BLK = 125
_EPT = _E // _NS
_NBLK = _EPT // _BLK
_RPT = _N // _NS
_ZR = 125
_DEGW = 16
_NRING = 5
_LOOK = 3
_DWIN = 16

_mesh = plsc.VectorSubcoreMesh(
    core_axis_name="c", subcore_axis_name="s", num_cores=_NC, num_subcores=_NS
)
_mesh1 = plsc.VectorSubcoreMesh(
    core_axis_name="c", subcore_axis_name="s", num_cores=1, num_subcores=_NS
)
_sc_params = pltpu.CompilerParams(use_tc_tiling_on_sc=False)


def _deg_body(dst_hbm, degp_hbm, dstv, onesv, zv, ssem, acc_sh):
  s = lax.axis_index("s")

  def _init(i, carry):
    onesv[i, :] = jnp.ones((_DEGW,), jnp.float32)
    zv[i, :] = jnp.zeros((_DEGW,), jnp.float32)
    return carry

  lax.fori_loop(0, _ZR, _init, 0)
  for r in range(_RPT // _ZR):
    pltpu.sync_copy(zv, acc_sh.at[pl.ds(s * _RPT + r * _ZR, _ZR)])
  pltpu.sync_copy(dst_hbm.at[s], dstv)
  plsc.subcore_barrier()

  def _step(j, carry):
    @pl.when(j >= _DWIN)
    def _():
      pltpu.make_async_copy(onesv, acc_sh.at[dstv.at[j]], ssem).wait()

    pltpu.async_copy(onesv, acc_sh.at[dstv.at[j]], ssem, add=True)
    return carry

  lax.fori_loop(0, _NBLK, _step, 0)

  def _drain(j, carry):
    pltpu.make_async_copy(onesv, acc_sh.at[dstv.at[j]], ssem).wait()
    return carry

  lax.fori_loop(0, _DWIN, _drain, 0)
  plsc.subcore_barrier()
  pltpu.sync_copy(acc_sh.at[pl.ds(s * _RPT, _RPT)],
                  degp_hbm.at[pl.ds(s * _RPT, _RPT)])


_deg_kernel = functools.partial(
    pl.kernel,
    out_type=jax.ShapeDtypeStruct((_N, _DEGW), jnp.float32),
    mesh=_mesh1,
    compiler_params=_sc_params,
    scratch_types=[
        pltpu.VMEM((_NBLK, _BLK), jnp.int32),
        pltpu.VMEM((_BLK, _DEGW), jnp.float32),
        pltpu.VMEM((_ZR, _DEGW), jnp.float32),
        pltpu.SemaphoreType.DMA,
        pltpu.VMEM_SHARED((_N, _DEGW), jnp.float32),
    ],
)(_deg_body)


def _make_agg(fh, nring, look):
  def _agg_body(tab_hbm, src_hbm, dst_hbm, aggp_hbm, srcv, dstv, rows, zv,
                *rest):
    srest = list(rest)
    gsems = tuple(srest[:nring])
    ssems = tuple(srest[nring:2 * nring])
    acc_sh = srest[2 * nring]
    c = lax.axis_index("c")
    s = lax.axis_index("s")

    def _zinit(i, carry):
      for f in range(fh // 16):
        zv[i, pl.ds(f * 16, 16)] = jnp.zeros((16,), jnp.float32)
      return carry

    lax.fori_loop(0, _ZR, _zinit, 0)
    for r in range(_RPT // _ZR):
      pltpu.sync_copy(zv, acc_sh.at[pl.ds(s * _RPT + r * _ZR, _ZR)])
    pltpu.sync_copy(src_hbm.at[s], srcv)
    pltpu.sync_copy(dst_hbm.at[s], dstv)
    plsc.subcore_barrier()

    sub_tab = tab_hbm.at[c]
    for b in range(look):
      pltpu.async_copy(sub_tab.at[srcv.at[b]], rows.at[b], gsems[b])

    def _outer(j4, carry):
      for b in range(nring):
        j = j4 * nring + b
        pltpu.make_async_copy(sub_tab.at[srcv.at[j]], rows.at[b],
                              gsems[b]).wait()
        pltpu.async_copy(rows.at[b], acc_sh.at[dstv.at[j]], ssems[b],
                         add=True)
        jn = j + look
        bn = (b + look) % nring

        @pl.when(jn < _NBLK)
        def _():
          @pl.when(jn >= nring)
          def _():
            pltpu.make_async_copy(rows.at[bn], acc_sh.at[dstv.at[jn]],
                                  ssems[bn]).wait()

          pltpu.async_copy(sub_tab.at[srcv.at[jn]], rows.at[bn],
                           gsems[bn])
      return carry

    lax.fori_loop(0, _NBLK // nring, _outer, 0)
    for b in range(nring):
      pltpu.make_async_copy(rows.at[b], acc_sh.at[dstv.at[0]],
                            ssems[b]).wait()
    plsc.subcore_barrier()
    pltpu.sync_copy(acc_sh.at[pl.ds(s * _RPT, _RPT)],
                    aggp_hbm.at[pl.ds(s * _RPT, _RPT), pl.ds(c * fh, fh)])

  return functools.partial(
      pl.kernel,
      out_type=jax.ShapeDtypeStruct((_N, 128), jnp.float32),
      mesh=_mesh,
      compiler_params=_sc_params,
      scratch_types=[
          pltpu.VMEM((_NBLK, _BLK), jnp.int32),
          pltpu.VMEM((_NBLK, _BLK), jnp.int32),
          pltpu.VMEM((nring, _BLK, fh), jnp.float32),
          pltpu.VMEM((_ZR, fh), jnp.float32),
      ] + [pltpu.SemaphoreType.DMA] * (2 * nring) + [
          pltpu.VMEM_SHARED((_N, fh), jnp.float32),
      ],
  )(_agg_body)


_agg1 = _make_agg(_FIN // 2, 5, 3)
_agg2 = _make_agg(_NCLS // 2, 10, 5)


_BN = 1000
_GN = _N // _BN
_FH1 = _FIN // 2
_FH2 = _NCLS // 2


def _dinv_from(degp_ref):
  deg = jnp.sum(degp_ref[...], axis=1) * (1.0 / _DEGW) + 1.0
  return lax.rsqrt(deg)[:, None]


def _prep_body(degp_ref, x_ref, xs2_ref):
  xs = x_ref[...] * _dinv_from(degp_ref)
  xs2_ref[0] = xs[:, :_FH1]
  xs2_ref[1] = xs[:, _FH1:]


_prep = pl.pallas_call(
    _prep_body,
    grid=(_GN,),
    in_specs=[
        pl.BlockSpec((_BN, _DEGW), lambda i: (i, 0)),
        pl.BlockSpec((_BN, _FIN), lambda i: (i, 0)),
    ],
    out_specs=pl.BlockSpec((_NC, _BN, _FH1), lambda i: (0, i, 0)),
    out_shape=jax.ShapeDtypeStruct((_NC, _N, _FH1), jnp.float32),
)


def _mid_body(degp_ref, aggp_ref, x_ref, w1_ref, b1_ref, w2_ref, z_ref,
              zs2_ref):
  dinv = _dinv_from(degp_ref)
  d2 = dinv * dinv
  ab = aggp_ref[...]
  y1a = dinv * ab[:, :_FH1] + d2 * x_ref[:, :_FH1]
  y1b = dinv * ab[:, _FH1:] + d2 * x_ref[:, _FH1:]
  h = (jnp.dot(y1a, w1_ref[:_FH1, :], preferred_element_type=jnp.float32)
       + jnp.dot(y1b, w1_ref[_FH1:, :], preferred_element_type=jnp.float32))
  h = jnp.maximum(h + b1_ref[...], 0.0)
  z = jnp.dot(h, w2_ref[...], preferred_element_type=jnp.float32)
  z_ref[...] = z
  zs = dinv * z
  zs2_ref[0] = zs[:, :_FH2]
  zs2_ref[1] = zs[:, _FH2:]


_mid = pl.pallas_call(
    _mid_body,
    grid=(_GN,),
    in_specs=[
        pl.BlockSpec((_BN, _DEGW), lambda i: (i, 0)),
        pl.BlockSpec((_BN, _FIN), lambda i: (i, 0)),
        pl.BlockSpec((_BN, _FIN), lambda i: (i, 0)),
        pl.BlockSpec((_FIN, _HID), lambda i: (0, 0)),
        pl.BlockSpec((1, _HID), lambda i: (0, 0)),
        pl.BlockSpec((_HID, _NCLS), lambda i: (0, 0)),
    ],
    out_specs=[
        pl.BlockSpec((_BN, _NCLS), lambda i: (i, 0)),
        pl.BlockSpec((_NC, _BN, _FH2), lambda i: (0, i, 0)),
    ],
    out_shape=[
        jax.ShapeDtypeStruct((_N, _NCLS), jnp.float32),
        jax.ShapeDtypeStruct((_NC, _N, _FH2), jnp.float32),
    ],
)


def _final_body(degp_ref, aggp_ref, z_ref, b2_ref, o_ref):
  dinv = _dinv_from(degp_ref)
  agg = aggp_ref[:, :_NCLS]
  y2 = dinv * agg + (dinv * dinv) * z_ref[...] + b2_ref[...]
  m = jnp.max(y2, axis=1, keepdims=True)
  lse = jnp.log(jnp.sum(jnp.exp(y2 - m), axis=1, keepdims=True)) + m
  o_ref[...] = y2 - lse


_final = pl.pallas_call(
    _final_body,
    grid=(_GN,),
    in_specs=[
        pl.BlockSpec((_BN, _DEGW), lambda i: (i, 0)),
        pl.BlockSpec((_BN, _FIN), lambda i: (i, 0)),
        pl.BlockSpec((_BN, _NCLS), lambda i: (i, 0)),
        pl.BlockSpec((1, _NCLS), lambda i: (0, 0)),
    ],
    out_specs=pl.BlockSpec((_BN, _NCLS), lambda i: (i, 0)),
    out_shape=jax.ShapeDtypeStruct((_N, _NCLS), jnp.float32),
)


def kernel(x, edge_index, W1, b1, W2, b2):
  ei = edge_index.astype(jnp.int32)
  src_a = ei[0].reshape(_NS, _NBLK, _BLK)
  dst_a = ei[1].reshape(_NS, _NBLK, _BLK)

  degp = _deg_kernel(dst_a)
  xs = _prep(degp, x)
  aggp1 = _agg1(xs, src_a, dst_a)
  z, zs = _mid(degp, aggp1, x, W1, b1.reshape(1, _HID), W2)
  aggp2 = _agg2(zs, src_a, dst_a)
  return _final(degp, aggp2, z, b2.reshape(1, _NCLS))
